# Initial kernel scaffold; baseline (speedup 1.0000x reference)
import jax
import jax.numpy as jnp
from jax.experimental import pallas as pl


def kernel(features, a_ids, b_ids, pretrained_embeddings, fresh_embeddings):
    raise NotImplementedError("write your pallas kernel here")


# speedup vs baseline: 8.7412x; 8.7412x over previous
import functools

import jax
import jax.numpy as jnp
from jax import lax
from jax.experimental import pallas as pl
from jax.experimental.pallas import tpu as pltpu
from jax.experimental.pallas import tpu_sc as plsc

VOCAB_FRESH = 100000
D = 32
B = 16384
L = 50
NF = 16

NC = 2
NS = 16
NW = NC * NS
ROWS_PER_W = B // NW
C = 16
IDS_PER_CHUNK = C * L
NCHUNK = ROWS_PER_W // C
OUT_W = NF + 2 * D
LANES = 16


def _reduce_rows(rows_ref, base):
    zero = jnp.zeros((LANES,), jnp.float32)

    def body(g, accs):
        a0, a1 = accs
        for u in range(5):
            r = base + g * 5 + u
            a0 = a0 + rows_ref[r, pl.ds(0, LANES)]
            a1 = a1 + rows_ref[r, pl.ds(LANES, LANES)]
        return (a0, a1)

    return lax.fori_loop(0, L // 5, body, (zero, zero))


def _sc_body(feat_hbm, aids_hbm, bids_hbm, pre_hbm, fresh_hbm, out_hbm,
             idx_a, fidx_a, idx_b, fidx_b,
             rows_pa, rows_fa, rows_pb, rows_fb,
             feat_v, out_v,
             sem_pa, sem_fa, sem_pb, sem_fb):
    wid = lax.axis_index("s") * NC + lax.axis_index("c")

    def chunk_body(g, carry):
        row0 = wid * ROWS_PER_W + g * C
        ids_off = row0 * L

        pltpu.sync_copy(aids_hbm.at[pl.ds(ids_off, IDS_PER_CHUNK)], idx_a)
        pltpu.sync_copy(bids_hbm.at[pl.ds(ids_off, IDS_PER_CHUNK)], idx_b)

        def mod_body(i, _):
            off = pl.multiple_of(i * LANES, 8)
            fidx_a[pl.ds(off, LANES)] = lax.rem(
                idx_a[pl.ds(off, LANES)], VOCAB_FRESH)
            fidx_b[pl.ds(off, LANES)] = lax.rem(
                idx_b[pl.ds(off, LANES)], VOCAB_FRESH)
            return 0
        lax.fori_loop(0, IDS_PER_CHUNK // LANES, mod_body, 0)

        cp_pa = pltpu.async_copy(pre_hbm.at[idx_a], rows_pa, sem_pa)
        cp_fa = pltpu.async_copy(fresh_hbm.at[fidx_a], rows_fa, sem_fa)
        cp_pb = pltpu.async_copy(pre_hbm.at[idx_b], rows_pb, sem_pb)
        cp_fb = pltpu.async_copy(fresh_hbm.at[fidx_b], rows_fb, sem_fb)

        pltpu.sync_copy(feat_hbm.at[pl.ds(row0, C)], feat_v)
        for b in range(C):
            out_v[b, pl.ds(0, NF)] = feat_v[b, :]

        scale = jnp.float32(1.0 / (2 * L))

        cp_pa.wait()
        cp_fa.wait()
        for b in range(C):
            p0, p1 = _reduce_rows(rows_pa, b * L)
            f0, f1 = _reduce_rows(rows_fa, b * L)
            out_v[b, pl.ds(NF, LANES)] = (p0 + f0) * scale
            out_v[b, pl.ds(NF + LANES, LANES)] = (p1 + f1) * scale

        cp_pb.wait()
        cp_fb.wait()
        for b in range(C):
            p0, p1 = _reduce_rows(rows_pb, b * L)
            f0, f1 = _reduce_rows(rows_fb, b * L)
            out_v[b, pl.ds(NF + D, LANES)] = (p0 + f0) * scale
            out_v[b, pl.ds(NF + D + LANES, LANES)] = (p1 + f1) * scale

        pltpu.sync_copy(out_v, out_hbm.at[pl.ds(row0, C)])
        return carry

    lax.fori_loop(0, NCHUNK, chunk_body, 0)


@jax.jit
def _run(features, a_flat, b_flat, pre, fresh):
    mesh = plsc.VectorSubcoreMesh(
        core_axis_name="c", subcore_axis_name="s",
        num_cores=NC, num_subcores=NS)
    fn = pl.kernel(
        _sc_body,
        out_type=jax.ShapeDtypeStruct((B, OUT_W), jnp.float32),
        mesh=mesh,
        compiler_params=pltpu.CompilerParams(use_tc_tiling_on_sc=False),
        scratch_types=[
            pltpu.VMEM((IDS_PER_CHUNK,), jnp.int32),
            pltpu.VMEM((IDS_PER_CHUNK,), jnp.int32),
            pltpu.VMEM((IDS_PER_CHUNK,), jnp.int32),
            pltpu.VMEM((IDS_PER_CHUNK,), jnp.int32),
            pltpu.VMEM((IDS_PER_CHUNK, D), jnp.float32),
            pltpu.VMEM((IDS_PER_CHUNK, D), jnp.float32),
            pltpu.VMEM((IDS_PER_CHUNK, D), jnp.float32),
            pltpu.VMEM((IDS_PER_CHUNK, D), jnp.float32),
            pltpu.VMEM((C, NF), jnp.float32),
            pltpu.VMEM((C, OUT_W), jnp.float32),
            pltpu.SemaphoreType.DMA,
            pltpu.SemaphoreType.DMA,
            pltpu.SemaphoreType.DMA,
            pltpu.SemaphoreType.DMA,
        ],
    )
    return fn(features, a_flat, b_flat, pre, fresh)


def kernel(features, a_ids, b_ids, pretrained_embeddings, fresh_embeddings):
    a_flat = a_ids.reshape(-1)
    b_flat = b_ids.reshape(-1)
    return _run(features, a_flat, b_flat,
                pretrained_embeddings, fresh_embeddings)


# speedup vs baseline: 12.7597x; 1.4597x over previous
import jax
import jax.numpy as jnp
from jax import lax
from jax.experimental import pallas as pl
from jax.experimental.pallas import tpu as pltpu
from jax.experimental.pallas import tpu_sc as plsc

VOCAB_FRESH = 100000
D = 32
B = 16384
L = 50
NF = 16

NC = 2
NS = 16
NW = NC * NS
ROWS_PER_W = B // NW
C = 16
IDS = C * L
NCHUNK = ROWS_PER_W // C
OUT_W = NF + 2 * D
LANES = 16
SCALE = 1.0 / (2 * L)


def _reduce_rows(rows_ref, base):
    zero = jnp.zeros((LANES,), jnp.float32)

    def body(g, accs):
        a0, a1 = accs
        for u in range(5):
            r = base + g * 5 + u
            a0 = a0 + rows_ref[r, pl.ds(0, LANES)]
            a1 = a1 + rows_ref[r, pl.ds(LANES, LANES)]
        return (a0, a1)

    return lax.fori_loop(0, L // 5, body, (zero, zero))


def _make_body(apply_mod, accumulate):
    def body(aux_hbm, aids_hbm, bids_hbm, tbl_hbm, out_hbm,
             ida0, ida1, idb0, idb1,
             rowsa0, rowsa1, rowsb0, rowsb1,
             outv0, outv1,
             sia0, sia1, sib0, sib1,
             sga0, sga1, sgb0, sgb1,
             sax0, sax1):
        ida = (ida0, ida1)
        idb = (idb0, idb1)
        rowsa = (rowsa0, rowsa1)
        rowsb = (rowsb0, rowsb1)
        outv = (outv0, outv1)
        sia = (sia0, sia1)
        sib = (sib0, sib1)
        sga = (sga0, sga1)
        sgb = (sgb0, sgb1)
        sax = (sax0, sax1)

        wid = lax.axis_index("s") * NC + lax.axis_index("c")
        row_base = wid * ROWS_PER_W

        def fetch(g, s):
            row0 = row_base + g * C
            off = row0 * L
            pltpu.async_copy(aids_hbm.at[pl.ds(off, IDS)], ida[s], sia[s])
            pltpu.async_copy(bids_hbm.at[pl.ds(off, IDS)], idb[s], sib[s])
            if accumulate:
                pltpu.async_copy(aux_hbm.at[pl.ds(row0, C)], outv[s], sax[s])
            else:
                pltpu.async_copy(
                    aux_hbm.at[pl.ds(row0, C)],
                    outv[s].at[:, pl.ds(0, NF)], sax[s])

        def mod_gather(g, s):
            pltpu.make_async_copy(
                aids_hbm.at[pl.ds(0, IDS)], ida[s], sia[s]).wait()
            pltpu.make_async_copy(
                bids_hbm.at[pl.ds(0, IDS)], idb[s], sib[s]).wait()
            if apply_mod:
                def mod_body(i, _):
                    off = pl.multiple_of(i * LANES, 8)
                    ida[s][pl.ds(off, LANES)] = lax.rem(
                        ida[s][pl.ds(off, LANES)], VOCAB_FRESH)
                    idb[s][pl.ds(off, LANES)] = lax.rem(
                        idb[s][pl.ds(off, LANES)], VOCAB_FRESH)
                    return 0
                lax.fori_loop(0, IDS // LANES, mod_body, 0)
            pltpu.async_copy(tbl_hbm.at[ida[s]], rowsa[s], sga[s])
            pltpu.async_copy(tbl_hbm.at[idb[s]], rowsb[s], sgb[s])

        def reduce_out(g, s):
            row0 = row_base + g * C
            pltpu.make_async_copy(
                tbl_hbm.at[ida[s]], rowsa[s], sga[s]).wait()
            pltpu.make_async_copy(
                tbl_hbm.at[idb[s]], rowsb[s], sgb[s]).wait()
            if accumulate:
                pltpu.make_async_copy(
                    aux_hbm.at[pl.ds(0, C)], outv[s], sax[s]).wait()
            else:
                pltpu.make_async_copy(
                    aux_hbm.at[pl.ds(0, C)],
                    outv[s].at[:, pl.ds(0, NF)], sax[s]).wait()
            scale = jnp.float32(SCALE)

            def red_body(b, _):
                a0, a1 = _reduce_rows(rowsa[s], b * L)
                b0, b1 = _reduce_rows(rowsb[s], b * L)
                if accumulate:
                    outv[s][b, pl.ds(NF, LANES)] = (
                        outv[s][b, pl.ds(NF, LANES)] + a0 * scale)
                    outv[s][b, pl.ds(NF + LANES, LANES)] = (
                        outv[s][b, pl.ds(NF + LANES, LANES)] + a1 * scale)
                    outv[s][b, pl.ds(NF + D, LANES)] = (
                        outv[s][b, pl.ds(NF + D, LANES)] + b0 * scale)
                    outv[s][b, pl.ds(NF + D + LANES, LANES)] = (
                        outv[s][b, pl.ds(NF + D + LANES, LANES)] + b1 * scale)
                else:
                    outv[s][b, pl.ds(NF, LANES)] = a0 * scale
                    outv[s][b, pl.ds(NF + LANES, LANES)] = a1 * scale
                    outv[s][b, pl.ds(NF + D, LANES)] = b0 * scale
                    outv[s][b, pl.ds(NF + D + LANES, LANES)] = b1 * scale
                return 0

            lax.fori_loop(0, C, red_body, 0)
            pltpu.sync_copy(outv[s], out_hbm.at[pl.ds(row0, C)])

        fetch(0, 0)
        mod_gather(0, 0)
        fetch(1, 1)
        mod_gather(1, 1)

        def pipe(j, _):
            g = j * 2
            reduce_out(g, 0)
            fetch(g + 2, 0)
            mod_gather(g + 2, 0)
            reduce_out(g + 1, 1)
            fetch(g + 3, 1)
            mod_gather(g + 3, 1)
            return 0

        lax.fori_loop(0, NCHUNK // 2 - 1, pipe, 0)
        reduce_out(NCHUNK - 2, 0)
        reduce_out(NCHUNK - 1, 1)

    return body


def _make_kernel(body):
    mesh = plsc.VectorSubcoreMesh(
        core_axis_name="c", subcore_axis_name="s",
        num_cores=NC, num_subcores=NS)
    return pl.kernel(
        body,
        out_type=jax.ShapeDtypeStruct((B, OUT_W), jnp.float32),
        mesh=mesh,
        compiler_params=pltpu.CompilerParams(use_tc_tiling_on_sc=False),
        scratch_types=(
            [pltpu.VMEM((IDS,), jnp.int32)] * 4
            + [pltpu.VMEM((IDS, D), jnp.float32)] * 4
            + [pltpu.VMEM((C, OUT_W), jnp.float32)] * 2
            + [pltpu.SemaphoreType.DMA] * 10
        ),
    )


@jax.jit
def _run(features, a_flat, b_flat, pre, fresh):
    fresh_fn = _make_kernel(_make_body(apply_mod=True, accumulate=False))
    pre_fn = _make_kernel(_make_body(apply_mod=False, accumulate=True))
    partial = fresh_fn(features, a_flat, b_flat, fresh)
    return pre_fn(partial, a_flat, b_flat, pre)


def kernel(features, a_ids, b_ids, pretrained_embeddings, fresh_embeddings):
    a_flat = a_ids.reshape(-1)
    b_flat = b_ids.reshape(-1)
    return _run(features, a_flat, b_flat,
                pretrained_embeddings, fresh_embeddings)
